# Initial kernel scaffold; baseline (speedup 1.0000x reference)
import jax
import jax.numpy as jnp
from jax.experimental import pallas as pl


def kernel(center, context, negatives, center_weight, context_weight):
    raise NotImplementedError("write your pallas kernel here")


# speedup vs baseline: 4.3784x; 4.3784x over previous
import functools

import jax
import jax.numpy as jnp
from jax import lax
from jax.experimental import pallas as pl
from jax.experimental.pallas import tpu as pltpu
from jax.experimental.pallas import tpu_sc as plsc

B = 16384
V = 1000000
D = 64
NNEG = 20
L = 16
DC = D // L
NC = 2
NS = 16
NW = NC * NS
TPW = B // NW
T = 16
NCHUNK = TPW // T
NR = T * NNEG
NG = NR // L


_DNUMS = lax.GatherDimensionNumbers(
    offset_dims=(), collapsed_slice_dims=(0,), start_index_map=(0,))


def _take(v, idx):
    return lax.gather(v, idx[:, None], _DNUMS, (1,),
                      mode=lax.GatherScatterMode.PROMISE_IN_BOUNDS)


def _assemble(partials, perms, masks):
    vecs = list(partials)
    for lvl, s in enumerate((1, 2, 4, 8)):
        perm, m = perms[lvl], masks[lvl]
        nxt = []
        for j in range(0, len(vecs), 2):
            a, b = vecs[j], vecs[j + 1]
            ta = a + _take(a, perm)
            tb = b + _take(b, perm)
            nxt.append(jnp.where(m, ta, tb))
        vecs = nxt
    return vecs[0]


def _sc_body(center_hbm, context_hbm, neg_hbm, cw_hbm, xw_hbm,
             pos_hbm, negout_hbm,
             cidx, xidx, nidx0, nidx1, nidx2,
             crow, xrow, nrow0, nrow1, nrow2,
             posb, negb, sem):
    cid = lax.axis_index("c")
    sid = lax.axis_index("s")
    wid = sid * NC + cid
    base = wid * TPW

    nrow_refs = ((nrow0, 0), (nrow1, 128), (nrow2, 256))

    def _nrow(r):
        for ref, off in reversed(nrow_refs):
            if r >= off:
                return ref, r - off
        raise AssertionError

    lane = lax.iota(jnp.int32, L)
    perms = [lane ^ s for s in (1, 2, 4, 8)]
    masks = [(lane & s) == 0 for s in (1, 2, 4, 8)]

    def chunk(c, carry):
        tb = base + c * T
        nb = tb * NNEG
        pltpu.sync_copy(center_hbm.at[pl.ds(tb, T)], cidx)
        pltpu.sync_copy(context_hbm.at[pl.ds(tb, T)], xidx)
        pltpu.sync_copy(neg_hbm.at[pl.ds(nb, 128)], nidx0)
        pltpu.sync_copy(neg_hbm.at[pl.ds(nb + 128, 128)], nidx1)
        pltpu.sync_copy(neg_hbm.at[pl.ds(nb + 256, 64)], nidx2)
        cps = [
            pltpu.async_copy(cw_hbm.at[cidx], crow, sem),
            pltpu.async_copy(xw_hbm.at[xidx], xrow, sem),
            pltpu.async_copy(xw_hbm.at[nidx0], nrow0, sem),
            pltpu.async_copy(xw_hbm.at[nidx1], nrow1, sem),
            pltpu.async_copy(xw_hbm.at[nidx2], nrow2, sem),
        ]
        for cp in cps:
            cp.wait()

        for g in range(NG):
            r0 = g * L
            cvec = {}
            partials = []
            for i in range(L):
                r = r0 + i
                t = r // NNEG
                if t not in cvec:
                    cvec[t] = [crow[t, pl.ds(dc * L, L)] for dc in range(DC)]
                nref, rr = _nrow(r)
                acc = nref[rr, pl.ds(0, L)] * cvec[t][0]
                for dc in range(1, DC):
                    acc = acc + nref[rr, pl.ds(dc * L, L)] * cvec[t][dc]
                partials.append(acc)
            negb[pl.ds(r0, L)] = _assemble(partials, perms, masks)

        partials = []
        for t in range(T):
            acc = crow[t, pl.ds(0, L)] * xrow[t, pl.ds(0, L)]
            for dc in range(1, DC):
                acc = acc + crow[t, pl.ds(dc * L, L)] * xrow[t, pl.ds(dc * L, L)]
            partials.append(acc)
        posb[:] = _assemble(partials, perms, masks)

        pltpu.sync_copy(posb, pos_hbm.at[pl.ds(tb, T)])
        pltpu.sync_copy(negb, negout_hbm.at[pl.ds(nb, NR)])
        return carry

    lax.fori_loop(0, NCHUNK, chunk, 0)


_sc_kernel = functools.partial(
    pl.kernel,
    out_type=[
        jax.ShapeDtypeStruct((B,), jnp.float32),
        jax.ShapeDtypeStruct((B * NNEG,), jnp.float32),
    ],
    mesh=plsc.VectorSubcoreMesh(core_axis_name="c", subcore_axis_name="s"),
    compiler_params=pltpu.CompilerParams(use_tc_tiling_on_sc=False),
    scratch_types=[
        pltpu.VMEM((T,), jnp.int32),
        pltpu.VMEM((T,), jnp.int32),
        pltpu.VMEM((128,), jnp.int32),
        pltpu.VMEM((128,), jnp.int32),
        pltpu.VMEM((64,), jnp.int32),
        pltpu.VMEM((T, D), jnp.float32),
        pltpu.VMEM((T, D), jnp.float32),
        pltpu.VMEM((128, D), jnp.float32),
        pltpu.VMEM((128, D), jnp.float32),
        pltpu.VMEM((64, D), jnp.float32),
        pltpu.VMEM((T,), jnp.float32),
        pltpu.VMEM((NR,), jnp.float32),
        pltpu.SemaphoreType.DMA,
    ],
)(_sc_body)


def kernel(center, context, negatives, center_weight, context_weight):
    negflat = negatives.reshape(-1).astype(jnp.int32)
    pos, negf = _sc_kernel(
        center.astype(jnp.int32),
        context.astype(jnp.int32),
        negflat,
        center_weight,
        context_weight,
    )
    return pos, negf.reshape(B, NNEG)


# speedup vs baseline: 4.9945x; 1.1407x over previous
import functools

import jax
import jax.numpy as jnp
from jax import lax
from jax.experimental import pallas as pl
from jax.experimental.pallas import tpu as pltpu
from jax.experimental.pallas import tpu_sc as plsc

B = 16384
V = 1000000
D = 64
NNEG = 20
L = 16
DC = D // L
NC = 2
NS = 16
NW = NC * NS
TPW = B // NW
T = 16
NCHUNK = TPW // T
NC2 = NCHUNK // 2
NR = T * NNEG
NG = NR // L

_DNUMS = lax.GatherDimensionNumbers(
    offset_dims=(), collapsed_slice_dims=(0,), start_index_map=(0,))


def _take(v, idx):
    return lax.gather(v, idx[:, None], _DNUMS, (1,),
                      mode=lax.GatherScatterMode.PROMISE_IN_BOUNDS)


def _assemble(partials, perms, masks):
    vecs = list(partials)
    for lvl in range(4):
        perm, m = perms[lvl], masks[lvl]
        nxt = []
        for j in range(0, len(vecs), 2):
            a, b = vecs[j], vecs[j + 1]
            ta = a + _take(a, perm)
            tb = b + _take(b, perm)
            nxt.append(jnp.where(m, ta, tb))
        vecs = nxt
    return vecs[0]


def _sc_body(center_hbm, context_hbm, neg_hbm, cw_hbm, xw_hbm,
             pos_hbm, negout_hbm,
             cidx, xidx, nidx,
             crowA, xrowA, nrow0A, nrow1A, nrow2A,
             crowB, xrowB, nrow0B, nrow1B, nrow2B,
             posb, negb, semA, semB):
    cid = lax.axis_index("c")
    sid = lax.axis_index("s")
    wid = sid * NC + cid
    base = wid * TPW

    bufsA = (crowA, xrowA, nrow0A, nrow1A, nrow2A)
    bufsB = (crowB, xrowB, nrow0B, nrow1B, nrow2B)

    lane = lax.iota(jnp.int32, L)
    perms = [lane ^ s for s in (1, 2, 4, 8)]
    masks = [(lane & s) == 0 for s in (1, 2, 4, 8)]

    pltpu.sync_copy(center_hbm.at[pl.ds(base, TPW)], cidx)
    pltpu.sync_copy(context_hbm.at[pl.ds(base, TPW)], xidx)
    pltpu.sync_copy(neg_hbm.at[pl.ds(base * NNEG, TPW * NNEG)], nidx)

    def issue(c, bufs, sem):
        crow, xrow, n0, n1, n2 = bufs
        o = c * T
        no = c * NR
        pltpu.async_copy(cw_hbm.at[cidx.at[pl.ds(o, T)]], crow, sem)
        pltpu.async_copy(xw_hbm.at[xidx.at[pl.ds(o, T)]], xrow, sem)
        pltpu.async_copy(xw_hbm.at[nidx.at[pl.ds(no, 128)]], n0, sem)
        pltpu.async_copy(xw_hbm.at[nidx.at[pl.ds(no + 128, 128)]], n1, sem)
        pltpu.async_copy(xw_hbm.at[nidx.at[pl.ds(no + 256, 64)]], n2, sem)

    def drain(bufs, sem):
        for d in bufs:
            n = d.shape[0]
            pltpu.make_async_copy(cw_hbm.at[pl.ds(0, n)], d, sem).wait()

    def compute(c, bufs):
        crow, xrow, n0, n1, n2 = bufs
        nrow_refs = ((n0, 0), (n1, 128), (n2, 256))

        def _nrow(r):
            for ref, off in reversed(nrow_refs):
                if r >= off:
                    return ref, r - off
            raise AssertionError

        nbase = c * NR
        for g in range(NG):
            r0 = g * L
            cvec = {}
            partials = []
            for i in range(L):
                r = r0 + i
                t = r // NNEG
                if t not in cvec:
                    cvec[t] = [crow[t, pl.ds(dc * L, L)] for dc in range(DC)]
                nref, rr = _nrow(r)
                acc = nref[rr, pl.ds(0, L)] * cvec[t][0]
                for dc in range(1, DC):
                    acc = acc + nref[rr, pl.ds(dc * L, L)] * cvec[t][dc]
                partials.append(acc)
            negb[pl.ds(nbase + r0, L)] = _assemble(partials, perms, masks)

        partials = []
        for t in range(T):
            acc = crow[t, pl.ds(0, L)] * xrow[t, pl.ds(0, L)]
            for dc in range(1, DC):
                acc = acc + crow[t, pl.ds(dc * L, L)] * xrow[t, pl.ds(dc * L, L)]
            partials.append(acc)
        posb[pl.ds(c * T, T)] = _assemble(partials, perms, masks)

    issue(0, bufsA, semA)

    def step(c2, carry):
        c0 = c2 * 2
        c1 = c0 + 1
        issue(c1, bufsB, semB)
        drain(bufsA, semA)
        compute(c0, bufsA)
        cnext = lax.rem(c0 + 2, NCHUNK)
        issue(cnext, bufsA, semA)
        drain(bufsB, semB)
        compute(c1, bufsB)
        return carry

    lax.fori_loop(0, NC2, step, 0)
    drain(bufsA, semA)

    pltpu.sync_copy(posb, pos_hbm.at[pl.ds(base, TPW)])
    pltpu.sync_copy(negb, negout_hbm.at[pl.ds(base * NNEG, TPW * NNEG)])


_sc_kernel = functools.partial(
    pl.kernel,
    out_type=[
        jax.ShapeDtypeStruct((B,), jnp.float32),
        jax.ShapeDtypeStruct((B * NNEG,), jnp.float32),
    ],
    mesh=plsc.VectorSubcoreMesh(core_axis_name="c", subcore_axis_name="s"),
    compiler_params=pltpu.CompilerParams(use_tc_tiling_on_sc=False),
    scratch_types=[
        pltpu.VMEM((TPW,), jnp.int32),
        pltpu.VMEM((TPW,), jnp.int32),
        pltpu.VMEM((TPW * NNEG,), jnp.int32),
        pltpu.VMEM((T, D), jnp.float32),
        pltpu.VMEM((T, D), jnp.float32),
        pltpu.VMEM((128, D), jnp.float32),
        pltpu.VMEM((128, D), jnp.float32),
        pltpu.VMEM((64, D), jnp.float32),
        pltpu.VMEM((T, D), jnp.float32),
        pltpu.VMEM((T, D), jnp.float32),
        pltpu.VMEM((128, D), jnp.float32),
        pltpu.VMEM((128, D), jnp.float32),
        pltpu.VMEM((64, D), jnp.float32),
        pltpu.VMEM((TPW,), jnp.float32),
        pltpu.VMEM((TPW * NNEG,), jnp.float32),
        pltpu.SemaphoreType.DMA,
        pltpu.SemaphoreType.DMA,
    ],
)(_sc_body)


def kernel(center, context, negatives, center_weight, context_weight):
    negflat = negatives.reshape(-1).astype(jnp.int32)
    pos, negf = _sc_kernel(
        center.astype(jnp.int32),
        context.astype(jnp.int32),
        negflat,
        center_weight,
        context_weight,
    )
    return pos, negf.reshape(B, NNEG)
